# baseline (device time: 17073 ns/iter reference)
import jax
import jax.numpy as jnp
from jax import lax
from jax.experimental import pallas as pl
from jax.experimental.pallas import tpu as pltpu

N_DEV = 8
M_PER = 128
M = 1024
N_COLS = 1024

_GROUPS = ((0, 384), (384, 384), (768, 256))


def _perm(order):
    out = []
    for np_ in range(8):
        bits = {a: (np_ >> (2 - i)) & 1 for i, a in enumerate(order)}
        x, y, z = bits["x"], bits["y"], bits["z"]
        out.append(4 * z + 2 * y + (x ^ y))
    return tuple(out)


_ORDERS = (("z", "y", "x"), ("y", "x", "z"), ("x", "z", "y"))
_PERMS = tuple(_perm(o) for o in _ORDERS)


def kernel(x, w_mat):
    bf = jnp.bfloat16
    f32 = jnp.float32

    def body(x_ref, w_ref, out_ref,
             p0, p1, p2, xg0, xg1, xg2, wb,
             r00, r01, r02, r10, r11, r12, r20, r21, r22,
             send_sems, recv_sems):
        p_refs = (p0, p1, p2)
        xg_refs = (xg0, xg1, xg2)
        rv_refs = ((r00, r01, r02), (r10, r11, r12), (r20, r21, r22))

        my = lax.axis_index("i")
        q = my % 4
        my_z = my // 4
        my_y = jnp.where(q >= 2, 1, 0)
        my_x = jnp.where((q == 1) | (q == 2), 1, 0)
        pz = my ^ 4
        py = my - q + (3 - q)
        px = my - q + (q ^ 1)

        coord = {"x": my_x, "y": my_y, "z": my_z}
        partner = {"x": px, "y": py, "z": pz}

        barrier_sem = pltpu.get_barrier_semaphore()
        for nbr in (pz, py, px):
            pl.semaphore_signal(
                barrier_sem, inc=1,
                device_id=(nbr,), device_id_type=pl.DeviceIdType.MESH,
            )

        wb[...] = w_ref[...].astype(bf)
        for g in range(3):
            for np_, cid in enumerate(_PERMS[g]):
                xg_refs[g][np_ * M_PER:(np_ + 1) * M_PER, :] = (
                    x_ref[cid * M_PER:(cid + 1) * M_PER, :].astype(bf)
                )

        cs = [[coord[a] for a in _ORDERS[g]] for g in range(3)]
        def start_rdma(g, sem, src_row0, rv_ref, rel0, nrows, axis):
            rdma = pltpu.make_async_remote_copy(
                src_ref=p_refs[g].at[pl.ds(src_row0, nrows), :],
                dst_ref=rv_ref.at[pl.ds(rel0, nrows), :],
                send_sem=send_sems.at[g, sem],
                recv_sem=recv_sems.at[g, sem],
                device_id=(partner[axis],),
                device_id_type=pl.DeviceIdType.MESH,
            )
            rdma.start()
            return rdma

        def acc(g, row0, rv_ref, rel0, nrows):
            p_refs[g][pl.ds(row0, nrows), :] = (
                p_refs[g][pl.ds(row0, nrows), :].astype(f32)
                + rv_ref[pl.ds(rel0, nrows), :].astype(f32)
            ).astype(bf)

        k0 = [cs[g][0] * 512 for g in range(3)]
        s0 = [(1 - cs[g][0]) * 512 for g in range(3)]
        p1_send = [k0[g] + (1 - cs[g][1]) * 256 for g in range(3)]
        prefix2 = [k0[g] + cs[g][1] * 256 for g in range(3)]
        p2_send = [prefix2[g] + (1 - cs[g][2]) * 128 for g in range(3)]
        fin = [prefix2[g] + cs[g][2] * 128 for g in range(3)]

        for g, (c0, nc) in enumerate(_GROUPS):
            p_refs[g][pl.ds(s0[g], 512), :] = jnp.dot(
                xg_refs[g][pl.ds(s0[g], 512), :], wb[:, c0:c0 + nc],
                preferred_element_type=f32,
            ).astype(bf)
        pl.semaphore_wait(barrier_sem, 3)
        p0subs = [[None] * 4 for _ in range(3)]
        for g in range(3):
            c1, c2 = cs[g][1], cs[g][2]
            rels = ((1 - c1) * 256 + (1 - c2) * 128,
                    (1 - c1) * 256 + c2 * 128,
                    c1 * 256 + (1 - c2) * 128,
                    c1 * 256 + c2 * 128)
            for i, rel in enumerate(rels):
                p0subs[g][i] = start_rdma(g, i, s0[g] + rel,
                                          rv_refs[g][0], rel, 128,
                                          _ORDERS[g][0])
        for g, (c0, nc) in enumerate(_GROUPS):
            p_refs[g][pl.ds(k0[g], 512), :] = jnp.dot(
                xg_refs[g][pl.ds(k0[g], 512), :], wb[:, c0:c0 + nc],
                preferred_element_type=f32,
            ).astype(bf)

        p1A, p1B = [None] * 3, [None] * 3
        for g in range(3):
            c1, c2 = cs[g][1], cs[g][2]
            p0subs[g][0].wait()
            acc(g, p1_send[g] + (1 - c2) * 128, rv_refs[g][0],
                (1 - c1) * 256 + (1 - c2) * 128, 128)
            p1A[g] = start_rdma(g, 4, p1_send[g] + (1 - c2) * 128,
                                rv_refs[g][1], (1 - c2) * 128, 128,
                                _ORDERS[g][1])
        for g in range(3):
            c1, c2 = cs[g][1], cs[g][2]
            p0subs[g][1].wait()
            acc(g, p1_send[g] + c2 * 128, rv_refs[g][0],
                (1 - c1) * 256 + c2 * 128, 128)
            p1B[g] = start_rdma(g, 5, p1_send[g] + c2 * 128,
                                rv_refs[g][1], c2 * 128, 128,
                                _ORDERS[g][1])
        for g in range(3):
            c1, c2 = cs[g][1], cs[g][2]
            p0subs[g][2].wait()
            acc(g, p2_send[g], rv_refs[g][0],
                c1 * 256 + (1 - c2) * 128, 128)

        p2r = [None] * 3
        for g in range(3):
            p1A[g].wait()
            acc(g, p2_send[g], rv_refs[g][1], (1 - cs[g][2]) * 128, 128)
            p2r[g] = start_rdma(g, 6, p2_send[g], rv_refs[g][2], 0, 128,
                                _ORDERS[g][2])
        for g in range(3):
            c1, c2 = cs[g][1], cs[g][2]
            p0subs[g][3].wait()
            acc(g, fin[g], rv_refs[g][0], c1 * 256 + c2 * 128, 128)
        for g in range(3):
            p1B[g].wait()
            acc(g, fin[g], rv_refs[g][1], cs[g][2] * 128, 128)

        for g, (c0, nc) in enumerate(_GROUPS):
            p2r[g].wait()
            out_ref[:, c0:c0 + nc] = (
                p_refs[g][pl.ds(fin[g], M_PER), :].astype(f32)
                + rv_refs[g][2][...].astype(f32)
            )

    scratch = [
        pltpu.VMEM((M, 384), bf),
        pltpu.VMEM((M, 384), bf),
        pltpu.VMEM((M, 256), bf),
        pltpu.VMEM((M, 128), bf),
        pltpu.VMEM((M, 128), bf),
        pltpu.VMEM((M, 128), bf),
        pltpu.VMEM((128, N_COLS), bf),
        pltpu.VMEM((512, 384), bf),
        pltpu.VMEM((256, 384), bf),
        pltpu.VMEM((128, 384), bf),
        pltpu.VMEM((512, 384), bf),
        pltpu.VMEM((256, 384), bf),
        pltpu.VMEM((128, 384), bf),
        pltpu.VMEM((512, 256), bf),
        pltpu.VMEM((256, 256), bf),
        pltpu.VMEM((128, 256), bf),
        pltpu.SemaphoreType.DMA((3, 7)),
        pltpu.SemaphoreType.DMA((3, 7)),
    ]
    return pl.pallas_call(
        body,
        out_shape=jax.ShapeDtypeStruct((M_PER, N_COLS), jnp.float32),
        in_specs=[
            pl.BlockSpec(memory_space=pltpu.VMEM),
            pl.BlockSpec(memory_space=pltpu.VMEM),
        ],
        out_specs=pl.BlockSpec(memory_space=pltpu.VMEM),
        scratch_shapes=scratch,
        compiler_params=pltpu.CompilerParams(collective_id=0),
    )(x, w_mat)
